# Initial kernel scaffold; baseline (speedup 1.0000x reference)
import jax
import jax.numpy as jnp
from jax.experimental import pallas as pl


def kernel(x, edge_index0, edge_index1, Wl0, Wr0, bl0, Wl1, Wr1, bl1):
    raise NotImplementedError("write your pallas kernel here")


# speedup vs baseline: 4.4452x; 4.4452x over previous
import functools

import jax
import jax.numpy as jnp
from jax import lax
from jax.experimental import pallas as pl
from jax.experimental.pallas import tpu as pltpu
from jax.experimental.pallas import tpu_sc as plsc

N0, N1, N2 = 10000, 5000, 1000
D_IN, D_HID, D_OUT = 128, 128, 41
E0, E1 = 320000, 80000

NC, NS = 2, 16
NW = NC * NS
CHUNK = 128

W0 = D_HID + 16
W1 = 48
BINS = 1024
TRASH = 1008

E0P = ((E0 + NW * CHUNK - 1) // (NW * CHUNK)) * (NW * CHUNK)
E1P = ((E1 + NW * CHUNK - 1) // (NW * CHUNK)) * (NW * CHUNK)


def _make_seg_sum(n_tab, width, chunks_per_tile):
  rpt = BINS // NS

  mesh = plsc.VectorSubcoreMesh(core_axis_name="c", subcore_axis_name="s",
                                num_cores=NC, num_subcores=NS)

  @functools.partial(
      pl.kernel,
      out_type=jax.ShapeDtypeStruct((NC * BINS, width), jnp.float32),
      mesh=mesh,
      compiler_params=pltpu.CompilerParams(use_tc_tiling_on_sc=False),
      scratch_types=[
          pltpu.VMEM((CHUNK,), jnp.int32),
          pltpu.VMEM((CHUNK,), jnp.int32),
          pltpu.VMEM((CHUNK, width), jnp.float32),
          pltpu.VMEM_SHARED((BINS, width), jnp.float32),
          pltpu.SemaphoreType.DMA,
      ],
  )
  def seg_sum(table_hbm, src_hbm, dst_hbm, zeros_hbm, out_hbm,
              src_v, dst_v, rows_v, accum_sp, sem):
    c = lax.axis_index("c")
    s = lax.axis_index("s")
    wid = s * NC + c
    pltpu.sync_copy(zeros_hbm.at[pl.ds(s * rpt, rpt)],
                    accum_sp.at[pl.ds(s * rpt, rpt)])
    plsc.subcore_barrier()

    base = wid * chunks_per_tile * CHUNK

    def body(i, carry):
      off = base + i * CHUNK
      pltpu.sync_copy(src_hbm.at[pl.ds(off, CHUNK)], src_v)
      pltpu.sync_copy(dst_hbm.at[pl.ds(off, CHUNK)], dst_v)
      pltpu.async_copy(table_hbm.at[src_v], rows_v, sem).wait()
      pltpu.sync_copy(rows_v, accum_sp.at[dst_v], add=True)
      return carry

    lax.fori_loop(0, chunks_per_tile, body, 0)
    plsc.subcore_barrier()
    pltpu.sync_copy(accum_sp.at[pl.ds(s * rpt, rpt)],
                    out_hbm.at[pl.ds(c * BINS + s * rpt, rpt)])

  return seg_sum


_seg_sum0 = _make_seg_sum(N1 + 8, W0, E0P // NW // CHUNK)
_seg_sum1 = _make_seg_sum(N2 + 8, W1, E1P // NW // CHUNK)


def _mm_body(x_ref, w_ref, o_ref):
  o_ref[...] = jnp.dot(x_ref[...], w_ref[...],
                       preferred_element_type=jnp.float32)


_mm = pl.pallas_call(
    _mm_body, out_shape=jax.ShapeDtypeStruct((N1, 2 * D_HID), jnp.float32))


def _layer0_post_body(parts_ref, t0_ref, bl0_ref, w1_ref, o_ref):
  s = parts_ref[0] + parts_ref[1]
  feat = s[:N2, :D_HID]
  cnt = s[:N2, D_HID:D_HID + 1]
  mean = feat / jnp.maximum(cnt, 1.0)
  h = jax.nn.relu(mean + bl0_ref[...] + t0_ref[...])
  o_ref[...] = jnp.dot(h, w1_ref[...], preferred_element_type=jnp.float32)


_layer0_post = pl.pallas_call(
    _layer0_post_body,
    out_shape=jax.ShapeDtypeStruct((N2, 2 * W1), jnp.float32))


def _final_body(parts_ref, t1_ref, bl1_ref, o_ref):
  s = parts_ref[0] + parts_ref[1]
  feat = s[:N2, :D_OUT]
  cnt = s[:N2, D_OUT:D_OUT + 1]
  o = feat / jnp.maximum(cnt, 1.0) + bl1_ref[...] + t1_ref[...]
  m = jnp.max(o, axis=-1, keepdims=True)
  lse = jnp.log(jnp.sum(jnp.exp(o - m), axis=-1, keepdims=True))
  o_ref[...] = o - m - lse


_final = pl.pallas_call(
    _final_body, out_shape=jax.ShapeDtypeStruct((N2, D_OUT), jnp.float32))


def kernel(x, edge_index0, edge_index1, Wl0, Wr0, bl0, Wl1, Wr1, bl1):
  f32 = jnp.float32

  yt = _mm(x[:N1], jnp.concatenate([Wl0, Wr0], axis=1))
  y0 = yt[:, :D_HID]
  t0 = yt[:N2, D_HID:]

  table0 = jnp.concatenate(
      [y0, jnp.ones((N1, 1), f32), jnp.zeros((N1, W0 - D_HID - 1), f32)],
      axis=1)
  table0 = jnp.pad(table0, ((0, 8), (0, 0)))
  src0 = jnp.pad(edge_index0[0], (0, E0P - E0))
  dst0 = jnp.pad(jnp.minimum(edge_index0[1], N2), (0, E0P - E0),
                 constant_values=TRASH)
  zeros0 = jnp.zeros((BINS, W0), f32)
  parts0 = _seg_sum0(table0, src0, dst0, zeros0).reshape(NC, BINS, W0)

  wl1p = jnp.pad(Wl1, ((0, 0), (0, W1 - D_OUT)))
  wr1p = jnp.pad(Wr1, ((0, 0), (0, W1 - D_OUT)))
  zt1 = _layer0_post(parts0, t0, bl0.reshape(1, D_HID),
                     jnp.concatenate([wl1p, wr1p], axis=1))

  table1 = jnp.concatenate(
      [zt1[:, :D_OUT], jnp.ones((N2, 1), f32),
       jnp.zeros((N2, W1 - D_OUT - 1), f32)], axis=1)
  table1 = jnp.pad(table1, ((0, 8), (0, 0)))
  src1 = jnp.pad(edge_index1[0], (0, E1P - E1))
  dst1 = jnp.pad(edge_index1[1], (0, E1P - E1), constant_values=TRASH)
  zeros1 = jnp.zeros((BINS, W1), f32)
  parts1 = _seg_sum1(table1, src1, dst1, zeros1).reshape(NC, BINS, W1)

  t1 = zt1[:, W1:W1 + D_OUT]
  bl1p = bl1.reshape(1, D_OUT)
  return _final(parts1, t1, bl1p)
